# Initial kernel scaffold; baseline (speedup 1.0000x reference)
import jax
import jax.numpy as jnp
from jax.experimental import pallas as pl


def kernel(query, memory, top_k):
    raise NotImplementedError("write your pallas kernel here")


# speedup vs baseline: 4.3566x; 4.3566x over previous
import functools

import jax
import jax.numpy as jnp
from jax import lax
from jax.experimental import pallas as pl
from jax.experimental.pallas import tpu as pltpu
from jax.experimental.pallas import tpu_sc as plsc

B = 8
D = 512
M = 65536
BM = 2048
NBLK = M // BM
LANES = 128
NCH = M // LANES
CPB = BM // LANES
K = 64
NEG = -3.0e38
EPS = 1e-8


def _sim_topk_kernel(q_ref, mem_ref, ts_ref, ti_ref, qm_ref, sims_ref, cmax_ref):
    i = pl.program_id(0)

    @pl.when(i == 0)
    def _init():
        qm_ref[:, :] = jnp.mean(q_ref[:, :, :], axis=1)

    qm = qm_ref[:, :]
    qn = jnp.maximum(jnp.sqrt(jnp.sum(qm * qm, axis=1, keepdims=True)), EPS)

    mem = mem_ref[0]
    dot = lax.dot_general(qm, mem, (((1,), (1,)), ((), ())),
                          preferred_element_type=jnp.float32,
                          precision=lax.Precision.HIGHEST)
    nb = jnp.maximum(jnp.sqrt(jnp.sum(mem * mem, axis=1)), EPS)
    nbb = nb.reshape(CPB, LANES)
    sim = dot.reshape(B, CPB, LANES) / (qn[:, :, None] * nbb[None, :, :])

    for b in range(B):
        sims_ref[pl.ds(i * CPB, CPB), pl.ds(b, 1), :] = sim[b].reshape(CPB, 1, LANES)

    @pl.when(i == NBLK - 1)
    def _topk():
        for s in range(NBLK):
            slab = sims_ref[pl.ds(s * CPB, CPB), :, :]
            cmax_ref[pl.ds(s * CPB, CPB), :] = jnp.max(slab, axis=2)

        lane_iota = lax.broadcasted_iota(jnp.int32, (B, LANES), 1)
        chunk_iota = lax.broadcasted_iota(jnp.int32, (NCH, B), 0)

        def body(k, carry):
            cm = cmax_ref[:, :]
            mval = jnp.max(cm, axis=0)
            cidx = jnp.argmax(cm, axis=0).astype(jnp.int32)
            rows = jnp.concatenate(
                [sims_ref[pl.ds(cidx[b], 1), pl.ds(b, 1), :].reshape(1, LANES)
                 for b in range(B)], axis=0)
            pos = jnp.argmax(rows, axis=1).astype(jnp.int32)
            gidx = cidx * LANES + pos
            ts_ref[pl.ds(k, 1), :] = mval.reshape(1, B)
            ti_ref[pl.ds(k, 1), :] = gidx.reshape(1, B)
            rows2 = jnp.where(lane_iota == pos[:, None], NEG, rows)
            for b in range(B):
                sims_ref[pl.ds(cidx[b], 1), pl.ds(b, 1), :] = rows2[b].reshape(1, 1, LANES)
            newmax = jnp.max(rows2, axis=1)
            cmax_ref[:, :] = jnp.where(chunk_iota == cidx[None, :], newmax[None, :], cm)
            return carry

        lax.fori_loop(0, K, body, 0)


def _sim_topk(query, memory):
    return pl.pallas_call(
        _sim_topk_kernel,
        grid=(NBLK,),
        in_specs=[
            pl.BlockSpec((B, 512, D), lambda i: (0, 0, 0)),
            pl.BlockSpec((1, BM, D), lambda i: (0, i, 0)),
        ],
        out_specs=[
            pl.BlockSpec((K, B), lambda i: (0, 0)),
            pl.BlockSpec((K, B), lambda i: (0, 0)),
        ],
        out_shape=[
            jax.ShapeDtypeStruct((K, B), jnp.float32),
            jax.ShapeDtypeStruct((K, B), jnp.int32),
        ],
        scratch_shapes=[
            pltpu.VMEM((B, D), jnp.float32),
            pltpu.VMEM((NCH, B, LANES), jnp.float32),
            pltpu.VMEM((NCH, B), jnp.float32),
        ],
    )(query, memory)


def _gather_sc(table, idx):
    info = plsc.get_sparse_core_info()
    NC, NS = info.num_cores, info.num_subcores
    NW = NC * NS
    n = idx.shape[0]
    bpw = n // NW
    mesh = plsc.VectorSubcoreMesh(core_axis_name="c", subcore_axis_name="s",
                                  num_cores=NC)

    @functools.partial(
        pl.kernel, mesh=mesh,
        out_type=jax.ShapeDtypeStruct((n, D), jnp.float32),
        scratch_types=[
            pltpu.VMEM((bpw,), jnp.int32),
            pltpu.VMEM((bpw, D), jnp.float32),
            pltpu.SemaphoreType.DMA,
        ],
    )
    def gk(table_hbm, idx_hbm, out_hbm, idx_v, rows_v, sem):
        wid = lax.axis_index("s") * NC + lax.axis_index("c")
        base = wid * bpw
        pltpu.sync_copy(idx_hbm.at[pl.ds(base, bpw)], idx_v)
        pltpu.async_copy(table_hbm.at[idx_v], rows_v, sem).wait()
        pltpu.sync_copy(rows_v, out_hbm.at[pl.ds(base, bpw)])

    return gk(table, idx)


def kernel(query, memory, top_k):
    ts, ti = _sim_topk(query, memory)
    top_sim = ts.T
    top_idx = ti.T
    rows = _gather_sc(memory[0], top_idx.reshape(-1))
    retrieved = rows.reshape(B, K, D)
    k_eff = jnp.minimum(jnp.asarray(top_k, jnp.int32), M)
    valid = jnp.arange(K, dtype=jnp.int32) < k_eff
    top_sim = jnp.where(valid[None, :], top_sim, 0.0)
    retrieved = jnp.where(valid[None, :, None], retrieved, 0.0)
    return retrieved, top_sim


# speedup vs baseline: 6.8068x; 1.5624x over previous
import functools

import jax
import jax.numpy as jnp
from jax import lax
from jax.experimental import pallas as pl
from jax.experimental.pallas import tpu as pltpu
from jax.experimental.pallas import tpu_sc as plsc

B = 8
D = 512
M = 65536
BM = 2048
NBLK = M // BM
LANES = 128
NCH = M // LANES
CPB = BM // LANES
K = 64
NEG = -3.0e38
EPS = 1e-8


def _split3(x):
    hi = x.astype(jnp.bfloat16)
    r = x - hi.astype(jnp.float32)
    mid = r.astype(jnp.bfloat16)
    lo = (r - mid.astype(jnp.float32)).astype(jnp.bfloat16)
    return hi, mid, lo


def _qprep_kernel(q_ref, q3_ref):
    qm = jnp.mean(q_ref[:, :, :], axis=1)
    qn = jnp.maximum(jnp.sqrt(jnp.sum(qm * qm, axis=1, keepdims=True)), EPS)
    qh = qm / qn
    hi, mid, lo = _split3(qh)
    q3_ref[:, :] = jnp.concatenate([hi, mid, lo], axis=0)


def _qprep(query):
    return pl.pallas_call(
        _qprep_kernel,
        out_shape=jax.ShapeDtypeStruct((3 * B, D), jnp.bfloat16),
    )(query)


def _sim_topk_kernel(q3_ref, mem_ref, ts_ref, ti_ref, sims_ref, cmax_ref):
    i = pl.program_id(0)

    q3 = q3_ref[:, :]
    mem = mem_ref[0]
    nb = jnp.maximum(jnp.sqrt(jnp.sum(mem * mem, axis=1, keepdims=True)), EPS)
    memn = mem / nb
    mhi, mmid, mlo = _split3(memn)
    nt = (((1,), (1,)), ((), ()))
    d3 = lax.dot_general(q3, mhi, nt, preferred_element_type=jnp.float32)
    d2 = lax.dot_general(q3[:2 * B], mmid, nt, preferred_element_type=jnp.float32)
    d1 = lax.dot_general(q3[:B], mlo, nt, preferred_element_type=jnp.float32)
    sim = ((d3[0:B] + d3[B:2 * B]) + (d3[2 * B:3 * B] + d2[0:B])
           + (d2[B:2 * B] + d1))

    for g in range(CPB):
        sims_ref[pl.ds(i * CPB + g, 1), :, :] = (
            sim[:, g * LANES:(g + 1) * LANES].reshape(1, B, LANES))

    @pl.when(i == NBLK - 1)
    def _topk():
        for s in range(NBLK):
            slab = sims_ref[pl.ds(s * CPB, CPB), :, :]
            cmax_ref[pl.ds(s * CPB, CPB), :] = jnp.max(slab, axis=2)

        lane_iota = lax.broadcasted_iota(jnp.int32, (B, LANES), 1)
        chunk_iota = lax.broadcasted_iota(jnp.int32, (NCH, B), 0)

        def body(k, carry):
            cm = cmax_ref[:, :]
            mval = jnp.max(cm, axis=0)
            cidx = jnp.argmax(cm, axis=0).astype(jnp.int32)
            rows = jnp.concatenate(
                [sims_ref[pl.ds(cidx[b], 1), pl.ds(b, 1), :].reshape(1, LANES)
                 for b in range(B)], axis=0)
            pos = jnp.argmax(rows, axis=1).astype(jnp.int32)
            gidx = cidx * LANES + pos
            ts_ref[pl.ds(k, 1), :] = mval.reshape(1, B)
            ti_ref[pl.ds(k, 1), :] = gidx.reshape(1, B)
            rows2 = jnp.where(lane_iota == pos[:, None], NEG, rows)
            for b in range(B):
                sims_ref[pl.ds(cidx[b], 1), pl.ds(b, 1), :] = rows2[b].reshape(1, 1, LANES)
            newmax = jnp.max(rows2, axis=1)
            cmax_ref[:, :] = jnp.where(chunk_iota == cidx[None, :], newmax[None, :], cm)
            return carry

        lax.fori_loop(0, K, body, 0)


def _sim_topk(q3, memory):
    return pl.pallas_call(
        _sim_topk_kernel,
        grid=(NBLK,),
        in_specs=[
            pl.BlockSpec((3 * B, D), lambda i: (0, 0)),
            pl.BlockSpec((1, BM, D), lambda i: (0, i, 0)),
        ],
        out_specs=[
            pl.BlockSpec((K, B), lambda i: (0, 0)),
            pl.BlockSpec((K, B), lambda i: (0, 0)),
        ],
        out_shape=[
            jax.ShapeDtypeStruct((K, B), jnp.float32),
            jax.ShapeDtypeStruct((K, B), jnp.int32),
        ],
        scratch_shapes=[
            pltpu.VMEM((NCH, B, LANES), jnp.float32),
            pltpu.VMEM((NCH, B), jnp.float32),
        ],
    )(q3, memory)


def _gather_sc(table, idx):
    info = plsc.get_sparse_core_info()
    NC, NS = info.num_cores, info.num_subcores
    NW = NC * NS
    n = idx.shape[0]
    bpw = n // NW
    mesh = plsc.VectorSubcoreMesh(core_axis_name="c", subcore_axis_name="s",
                                  num_cores=NC)

    @functools.partial(
        pl.kernel, mesh=mesh,
        out_type=jax.ShapeDtypeStruct((n, D), jnp.float32),
        scratch_types=[
            pltpu.VMEM((bpw,), jnp.int32),
            pltpu.VMEM((bpw, D), jnp.float32),
            pltpu.SemaphoreType.DMA,
        ],
    )
    def gk(table_hbm, idx_hbm, out_hbm, idx_v, rows_v, sem):
        wid = lax.axis_index("s") * NC + lax.axis_index("c")
        base = wid * bpw
        pltpu.sync_copy(idx_hbm.at[pl.ds(base, bpw)], idx_v)
        pltpu.async_copy(table_hbm.at[idx_v], rows_v, sem).wait()
        pltpu.sync_copy(rows_v, out_hbm.at[pl.ds(base, bpw)])

    return gk(table, idx)


def kernel(query, memory, top_k):
    q3 = _qprep(query)
    ts, ti = _sim_topk(q3, memory)
    top_sim = ts.T
    top_idx = ti.T
    rows = _gather_sc(memory[0], top_idx.reshape(-1))
    retrieved = rows.reshape(B, K, D)
    k_eff = jnp.minimum(jnp.asarray(top_k, jnp.int32), M)
    valid = jnp.arange(K, dtype=jnp.int32) < k_eff
    top_sim = jnp.where(valid[None, :], top_sim, 0.0)
    retrieved = jnp.where(valid[None, :, None], retrieved, 0.0)
    return retrieved, top_sim
